# baseline (speedup 1.0000x reference)
import functools

import jax
import jax.numpy as jnp
from jax import lax
from jax.experimental import pallas as pl
from jax.experimental.pallas import tpu as pltpu
from jax.experimental.pallas import tpu_sc as plsc

_NUM_CORES = 2
_NUM_SUBCORES = 16
_NW = _NUM_CORES * _NUM_SUBCORES
_LANES = 16
_NBUF = 3


@functools.lru_cache(maxsize=None)
def _make_gather(batch: int, seq: int, vocab: int, dim: int):
    assert batch % _NW == 0
    b_per_w = batch // _NW
    n_blk = b_per_w // _LANES
    assert dim == 32
    halves = dim // _LANES
    n_body = (seq - 2) // _NBUF
    tail = seq - n_body * _NBUF

    mesh = plsc.VectorSubcoreMesh(core_axis_name="c", subcore_axis_name="s")

    @functools.partial(
        pl.kernel,
        mesh=mesh,
        compiler_params=pltpu.CompilerParams(
            use_tc_tiling_on_sc=False, needs_layout_passes=False
        ),
        out_type=jax.ShapeDtypeStruct((seq, dim, batch), jnp.float32),
        scratch_types=[
            pltpu.VMEM((seq, b_per_w), jnp.int32),
            *( [pltpu.VMEM((halves * b_per_w,), jnp.int32)] * _NBUF ),
            *( [pltpu.VMEM((halves * b_per_w, _LANES), jnp.float32)] * _NBUF ),
            *( [pltpu.VMEM((dim, b_per_w + 1), jnp.float32)] * _NBUF ),
            *( [pltpu.SemaphoreType.DMA] * (2 * _NBUF) ),
        ],
    )
    def gather_kernel(table_hbm, idx_hbm, out_hbm, idx_all, *bufs):
        didx_bufs = bufs[0:_NBUF]
        rows = bufs[_NBUF:2 * _NBUF]
        tvs = bufs[2 * _NBUF:3 * _NBUF]
        gsems = bufs[3 * _NBUF:4 * _NBUF]
        wsems = bufs[4 * _NBUF:5 * _NBUF]

        wid = lax.axis_index("s") * _NUM_CORES + lax.axis_index("c")
        b0 = wid * b_per_w

        pltpu.sync_copy(idx_hbm.at[:, pl.ds(b0, b_per_w)], idx_all)

        def start(l, k):
            def bld(jb, c):
                v = idx_all[l, pl.ds(jb * _LANES, _LANES)] * halves
                for h in range(halves):
                    didx_bufs[k][pl.ds(h * b_per_w + jb * _LANES, _LANES)] = v + h
                return c

            lax.fori_loop(0, n_blk, bld, 0, unroll=4)
            pltpu.async_copy(table_hbm.at[didx_bufs[k]], rows[k], gsems[k])

        def wait_gather(l, k):
            pltpu.make_async_copy(
                table_hbm.at[didx_bufs[k]], rows[k], gsems[k]
            ).wait()

        def wait_write(l, k):
            pltpu.make_async_copy(
                tvs[k].at[:, pl.ds(0, b_per_w)],
                out_hbm.at[l, :, pl.ds(b0, b_per_w)],
                wsems[k],
            ).wait()

        def compute_and_write(l, k):
            iota = lax.iota(jnp.int32, _LANES)

            def tr(jb, c):
                for rr in range(8):
                    r = jb * 8 + rr
                    rsplat = jnp.full((_LANES,), r, jnp.int32)
                    for h in range(halves):
                        v = rows[k][h * b_per_w + r, :]
                        plsc.store_scatter(tvs[k], [iota + h * _LANES, rsplat], v)
                return c

            lax.fori_loop(0, b_per_w // 8, tr, 0, unroll=2)

            pltpu.async_copy(
                tvs[k].at[:, pl.ds(0, b_per_w)],
                out_hbm.at[l, :, pl.ds(b0, b_per_w)],
                wsems[k],
            )

        start(0, 0)
        start(1, 1)

        def body(i, carry):
            for k in range(_NBUF):
                l = _NBUF * i + k
                start(l + 2, (k + 2) % _NBUF)
                wait_gather(l, k)

                @pl.when(i >= 1)
                def _():
                    wait_write(l, k)

                compute_and_write(l, k)
            return carry

        lax.fori_loop(0, n_body, body, 0)

        for t in range(tail):
            l = _NBUF * n_body + t
            k = l % _NBUF
            wait_gather(l, k)
            if n_body >= 1:
                wait_write(l, k)
            compute_and_write(l, k)

        for k in range(_NBUF):
            pltpu.make_async_copy(
                tvs[k].at[:, pl.ds(0, b_per_w)],
                out_hbm.at[0, :, pl.ds(b0, b_per_w)],
                wsems[k],
            ).wait()

    return gather_kernel


def kernel(inputs, table):
    b, l = inputs.shape
    vocab, dim = table.shape
    idx_lmajor = inputs.T
    tbl16 = table.reshape(vocab * dim // _LANES, _LANES)
    out = _make_gather(b, l, vocab, dim)(tbl16, idx_lmajor)
    return out.transpose(2, 0, 1)
